# baseline (speedup 1.0000x reference)
import functools

import jax
import jax.numpy as jnp
from jax import lax
from jax.experimental import pallas as pl
from jax.experimental.pallas import tpu as pltpu
from jax.experimental.pallas import tpu_sc as plsc

H = 16
W = 16
HW = H * W
B = 4
C = 16
BC = B * C
N = 262144
EPS = 1e-6
T0 = -1.0 - EPS
INV = H / (2.0 + 2 * EPS)

NC = 1
NS = 16
NW = NC * NS
PTS = N // NW
CHUNK = 64
NCHUNK = PTS // CHUNK
GROUPS = CHUNK // 16
SL = HW * BC // NS

_mesh = plsc.VectorSubcoreMesh(core_axis_name="c", subcore_axis_name="s", num_cores=1)


@functools.partial(
    pl.kernel,
    out_type=(
        jax.ShapeDtypeStruct((NC, 128, 128), jnp.float32),
        jax.ShapeDtypeStruct((NC, NS, HW), jnp.float32),
    ),
    mesh=_mesh,
    compiler_params=pltpu.CompilerParams(needs_layout_passes=False),
    scratch_types=[
        pltpu.VMEM((2, CHUNK, 2), jnp.float32),
        pltpu.VMEM((2, B, CHUNK, C), jnp.float32),
        pltpu.VMEM((HW * BC,), jnp.float32),
        pltpu.VMEM((16 * HW,), jnp.float32),
        pltpu.VMEM((128, 128), jnp.float32),
        pltpu.VMEM((8, 128), jnp.float32),
        pltpu.VMEM((8, 128), jnp.float32),
        pltpu.VMEM((HW,), jnp.float32),
        pltpu.VMEM((128,), jnp.int32),
        pltpu.VMEM_SHARED((128, 128), jnp.float32),
        pltpu.SemaphoreType.DMA,
        pltpu.SemaphoreType.DMA,
    ],
)
def _pool_sc(coords_hbm, values_hbm, out_sums, out_cnts,
             cbuf, vbuf, acc, cntacc, acc2d, zbuf, red, cnt256, idxref,
             shared2, sem0, sem1):
    cid = lax.axis_index("c")
    sid = lax.axis_index("s")
    wid = cid * NS + sid
    base = wid * PTS

    iota = lax.broadcasted_iota(jnp.int32, (16,), 0)
    zeros = jnp.zeros((16,), jnp.float32)
    ones = jnp.ones((16,), jnp.float32)

    def zero_acc(i, _):
        acc[pl.ds(i * 16, 16)] = zeros
        return 0
    lax.fori_loop(0, HW * BC // 16, zero_acc, 0)

    def zero_cnt(i, _):
        cntacc[pl.ds(i * 16, 16)] = zeros
        return 0
    lax.fori_loop(0, 16 * HW // 16, zero_cnt, 0)

    sems = (sem0, sem1)

    def start(par, k):
        off = base + k * CHUNK
        pltpu.make_async_copy(
            coords_hbm.at[pl.ds(off, CHUNK)], cbuf.at[par], sems[par]).start()
        for b in range(B):
            pltpu.make_async_copy(
                values_hbm.at[b, pl.ds(off, CHUNK)], vbuf.at[par, b],
                sems[par]).start()

    def wait(par):
        pltpu.make_async_copy(
            coords_hbm.at[pl.ds(0, CHUNK)], cbuf.at[par], sems[par]).wait()
        for b in range(B):
            pltpu.make_async_copy(
                values_hbm.at[b, pl.ds(0, CHUNK)], vbuf.at[par, b],
                sems[par]).wait()

    def compute(par):
        def group_body(g, _):
            rows = g * 16 + iota
            x = plsc.load_gather(cbuf.at[par], [rows, jnp.zeros((16,), jnp.int32)])
            y = plsc.load_gather(cbuf.at[par], [rows, jnp.ones((16,), jnp.int32)])
            bx = ((x - T0) * INV).astype(jnp.int32)
            by = ((y - T0) * INV).astype(jnp.int32)
            binv = bx + by * H
            plsc.addupdate_scatter(cntacc, [iota * HW + binv], ones)
            bofs = binv * BC
            for l in range(16):
                idx0 = iota + bofs[l]
                p = g * 16 + l
                for b in range(B):
                    v = vbuf[par, b, p]
                    plsc.addupdate_scatter(acc, [idx0 + b * C], v)
            return 0
        lax.fori_loop(0, GROUPS, group_body, 0)

    start(0, 0)

    def chunk_pair(j, _):
        k0 = 2 * j
        start(1, k0 + 1)
        wait(0)
        compute(0)

        @pl.when(k0 + 2 < NCHUNK)
        def _():
            start(0, k0 + 2)
        wait(1)
        compute(1)
        return 0

    lax.fori_loop(0, NCHUNK // 2, chunk_pair, 0)

    def cnt_red(j, _):
        s = zeros
        for l in range(16):
            s = s + cntacc[pl.ds(l * HW + j * 16, 16)]
        cnt256[pl.ds(j * 16, 16)] = s
        return 0
    lax.fori_loop(0, HW // 16, cnt_red, 0)
    pltpu.sync_copy(cnt256, out_cnts.at[cid, sid])

    def row_copy(r, _):
        for jj in range(8):
            acc2d[r, pl.ds(jj * 16, 16)] = acc[pl.ds(r * 128 + jj * 16, 16)]
        return 0
    lax.fori_loop(0, 128, row_copy, 0)

    for r in range(8):
        for jj in range(8):
            zbuf[r, pl.ds(jj * 16, 16)] = zeros
    pltpu.sync_copy(zbuf, shared2.at[pl.ds(sid * 8, 8)])

    def idx_fill(g, _):
        idxref[pl.ds(g * 16, 16)] = g * 16 + iota
        return 0
    lax.fori_loop(0, 8, idx_fill, 0)

    plsc.subcore_barrier()
    pltpu.sync_copy(acc2d, shared2.at[idxref], add=True)
    plsc.subcore_barrier()

    pltpu.sync_copy(shared2.at[pl.ds(sid * 8, 8)], red)
    pltpu.sync_copy(red, out_sums.at[cid, pl.ds(sid * 8, 8)])


def kernel(coords, values):
    sums2, cnts = _pool_sc(coords, values)
    sums = (sums2[0] + sums2[1]).reshape(HW, B, C)
    cnt = cnts.sum(axis=(0, 1))
    means = sums / cnt[:, None, None]
    return means.transpose(1, 0, 2).reshape(B, HW * C)


# speedup vs baseline: 5.5361x; 5.5361x over previous
import functools

import jax
import jax.numpy as jnp
from jax import lax
from jax.experimental import pallas as pl
from jax.experimental.pallas import tpu as pltpu
from jax.experimental.pallas import tpu_sc as plsc

H = 16
W = 16
HW = H * W
B = 4
C = 16
BC = B * C
N = 262144
EPS = 1e-6
T0 = -1.0 - EPS
INV = H / (2.0 + 2 * EPS)

NC = 2
NS = 16
NW = NC * NS
PTS = N // NW
CHUNK = 512
NCHUNK = PTS // CHUNK
GROUPS = CHUNK // 16
SL = HW * BC // NS

_mesh = plsc.VectorSubcoreMesh(core_axis_name="c", subcore_axis_name="s")


@functools.partial(
    pl.kernel,
    out_type=(
        jax.ShapeDtypeStruct((NC, 128, 128), jnp.float32),
        jax.ShapeDtypeStruct((NC, NS, HW), jnp.float32),
    ),
    mesh=_mesh,
    compiler_params=pltpu.CompilerParams(needs_layout_passes=False),
    scratch_types=[
        pltpu.VMEM((2, 2, CHUNK), jnp.float32),
        pltpu.VMEM((2, B, C, CHUNK), jnp.float32),
        pltpu.VMEM((HW * BC,), jnp.float32),
        pltpu.VMEM((16 * HW,), jnp.float32),
        pltpu.VMEM((128, 128), jnp.float32),
        pltpu.VMEM((8, 128), jnp.float32),
        pltpu.VMEM((8, 128), jnp.float32),
        pltpu.VMEM((HW,), jnp.float32),
        pltpu.VMEM((128,), jnp.int32),
        pltpu.VMEM_SHARED((128, 128), jnp.float32),
        pltpu.SemaphoreType.DMA,
        pltpu.SemaphoreType.DMA,
    ],
)
def _pool_sc(coords_hbm, values_hbm, out_sums, out_cnts,
             cbuf, vbuf, acc, cntacc, acc2d, zbuf, red, cnt256, idxref,
             shared2, sem0, sem1):
    cid = lax.axis_index("c")
    sid = lax.axis_index("s")
    wid = cid * NS + sid
    base = wid * PTS

    iota = lax.broadcasted_iota(jnp.int32, (16,), 0)
    zeros = jnp.zeros((16,), jnp.float32)
    ones = jnp.ones((16,), jnp.float32)

    def zero_acc(i, _):
        acc[pl.ds(i * 16, 16)] = zeros
        return 0
    lax.fori_loop(0, HW * BC // 16, zero_acc, 0)

    def zero_cnt(i, _):
        cntacc[pl.ds(i * 16, 16)] = zeros
        return 0
    lax.fori_loop(0, 16 * HW // 16, zero_cnt, 0)

    sems = (sem0, sem1)

    cvecs = [(iota + k) & 15 for k in range(16)]

    def start(par, k):
        off = base + k * CHUNK
        pltpu.make_async_copy(
            coords_hbm.at[:, pl.ds(off, CHUNK)], cbuf.at[par], sems[par]).start()
        pltpu.make_async_copy(
            values_hbm.at[:, :, pl.ds(off, CHUNK)], vbuf.at[par],
            sems[par]).start()

    def wait(par):
        pltpu.make_async_copy(
            coords_hbm.at[:, pl.ds(0, CHUNK)], cbuf.at[par], sems[par]).wait()
        pltpu.make_async_copy(
            values_hbm.at[:, :, pl.ds(0, CHUNK)], vbuf.at[par],
            sems[par]).wait()

    def compute(par):
        def group_body(g, _):
            p0 = g * 16
            x = cbuf[par, 0, pl.ds(p0, 16)]
            y = cbuf[par, 1, pl.ds(p0, 16)]
            bx = ((x - T0) * INV).astype(jnp.int32)
            by = ((y - T0) * INV).astype(jnp.int32)
            binv = bx + by * H
            plsc.addupdate_scatter(cntacc, [iota * HW + binv], ones)
            bofs = binv * BC
            pvec = iota + p0
            for b in range(B):
                sb = bofs + b * C
                for k in range(16):
                    v = plsc.load_gather(vbuf.at[par, b], [cvecs[k], pvec])
                    plsc.addupdate_scatter(acc, [sb + cvecs[k]], v)
            return 0
        lax.fori_loop(0, GROUPS, group_body, 0)

    start(0, 0)

    def chunk_pair(j, _):
        k0 = 2 * j
        start(1, k0 + 1)
        wait(0)
        compute(0)

        @pl.when(k0 + 2 < NCHUNK)
        def _():
            start(0, k0 + 2)
        wait(1)
        compute(1)
        return 0

    lax.fori_loop(0, NCHUNK // 2, chunk_pair, 0)

    def cnt_red(j, _):
        s = zeros
        for l in range(16):
            s = s + cntacc[pl.ds(l * HW + j * 16, 16)]
        cnt256[pl.ds(j * 16, 16)] = s
        return 0
    lax.fori_loop(0, HW // 16, cnt_red, 0)
    pltpu.sync_copy(cnt256, out_cnts.at[cid, sid])

    def row_copy(r, _):
        for jj in range(8):
            acc2d[r, pl.ds(jj * 16, 16)] = acc[pl.ds(r * 128 + jj * 16, 16)]
        return 0
    lax.fori_loop(0, 128, row_copy, 0)

    for r in range(8):
        for jj in range(8):
            zbuf[r, pl.ds(jj * 16, 16)] = zeros
    pltpu.sync_copy(zbuf, shared2.at[pl.ds(sid * 8, 8)])

    def idx_fill(g, _):
        idxref[pl.ds(g * 16, 16)] = g * 16 + iota
        return 0
    lax.fori_loop(0, 8, idx_fill, 0)

    plsc.subcore_barrier()
    pltpu.sync_copy(acc2d, shared2.at[idxref], add=True)
    plsc.subcore_barrier()

    pltpu.sync_copy(shared2.at[pl.ds(sid * 8, 8)], red)
    pltpu.sync_copy(red, out_sums.at[cid, pl.ds(sid * 8, 8)])


def kernel(coords, values):
    sums2, cnts = _pool_sc(coords.T, values.transpose(0, 2, 1))
    sums = (sums2[0] + sums2[1]).reshape(HW, B, C)
    cnt = cnts.sum(axis=(0, 1))
    means = sums / cnt[:, None, None]
    return means.transpose(1, 0, 2).reshape(B, HW * C)
